# Initial kernel scaffold; baseline (speedup 1.0000x reference)
import jax
import jax.numpy as jnp
from jax.experimental import pallas as pl


def kernel(x, edge_index, W1_l, b1, W1_r, W2_l, b2, W2_r):
    raise NotImplementedError("write your pallas kernel here")


# speedup vs baseline: 8.3605x; 8.3605x over previous
import functools

import jax
import jax.numpy as jnp
from jax import lax
from jax.experimental import pallas as pl
from jax.experimental.pallas import tpu as pltpu
from jax.experimental.pallas import tpu_sc as plsc

N = 10000
E = 320000
D_IN = 128
D_HID = 64
NC = 2
NS = 16
NW = NC * NS
CH = 128
CPW = 79
EPAD = NW * CPW * CH
NPAD = 10240
RPT = NPAD // NS
RB = 2048


def _mm2_body(x_ref, wl_ref, wr_ref, b_ref, y_ref, z_ref):
    xb = x_ref[...]
    y_ref[...] = jnp.dot(xb, wl_ref[...], preferred_element_type=jnp.float32)
    z_ref[...] = (jnp.dot(xb, wr_ref[...], preferred_element_type=jnp.float32)
                  + b_ref[...])


def _phase_a(x_pad, W1_l, W1_r, b1_2d):
    return pl.pallas_call(
        _mm2_body,
        grid=(NPAD // RB,),
        in_specs=[
            pl.BlockSpec((RB, D_IN), lambda i: (i, 0)),
            pl.BlockSpec((D_IN, D_HID), lambda i: (0, 0)),
            pl.BlockSpec((D_IN, D_HID), lambda i: (0, 0)),
            pl.BlockSpec((1, D_HID), lambda i: (0, 0)),
        ],
        out_specs=[
            pl.BlockSpec((RB, D_HID), lambda i: (i, 0)),
            pl.BlockSpec((RB, D_HID), lambda i: (i, 0)),
        ],
        out_shape=[
            jax.ShapeDtypeStruct((NPAD, D_HID), jnp.float32),
            jax.ShapeDtypeStruct((NPAD, D_HID), jnp.float32),
        ],
    )(x_pad, W1_l, W1_r, b1_2d)


def _sc_agg64_body(ytab, srcm, dstm, agg_out, cnt_out,
                   idx_s, idx_d, rows, ones_v, zcnt, agg_sh, cnt_sh, sem):
    c = lax.axis_index("c")
    s = lax.axis_index("s")
    zero16 = jnp.zeros((16,), jnp.float32)
    one16 = jnp.ones((16,), jnp.float32)

    def zrow_body(i, carry):
        for j in range(D_HID // 16):
            rows[i, pl.ds(j * 16, 16)] = zero16
        return carry

    lax.fori_loop(0, CH, zrow_body, 0)

    def fill_body(i, carry):
        ones_v[pl.ds(i * 16, 16)] = one16
        return carry

    lax.fori_loop(0, CH // 16, fill_body, 0)

    def zcnt_body(i, carry):
        zcnt[pl.ds(i * 16, 16)] = zero16
        return carry

    lax.fori_loop(0, RPT // 16, zcnt_body, 0)

    for k in range(RPT // CH):
        pltpu.sync_copy(rows, agg_sh.at[pl.ds(s * RPT + k * CH, CH)])
    pltpu.sync_copy(zcnt, cnt_sh.at[pl.ds(s * RPT, RPT)])
    plsc.subcore_barrier()

    base = (c * NS + s) * CPW

    def edge_body(i, carry):
        pltpu.sync_copy(srcm.at[base + i], idx_s)
        pltpu.sync_copy(dstm.at[base + i], idx_d)
        pltpu.async_copy(ytab.at[idx_s], rows, sem).wait()
        pltpu.sync_copy(rows, agg_sh.at[idx_d], add=True)
        pltpu.sync_copy(ones_v, cnt_sh.at[idx_d], add=True)
        return carry

    lax.fori_loop(0, CPW, edge_body, 0)
    plsc.subcore_barrier()

    pltpu.sync_copy(agg_sh.at[pl.ds(s * RPT, RPT)], agg_out.at[c, s])
    pltpu.sync_copy(cnt_sh.at[pl.ds(s * RPT, RPT)], cnt_out.at[c, s])


def _phase_b(ytab, srcm, dstm):
    mesh = plsc.VectorSubcoreMesh(core_axis_name="c", subcore_axis_name="s")
    f = functools.partial(
        pl.kernel,
        out_type=[
            jax.ShapeDtypeStruct((NC, NS, RPT, D_HID), jnp.float32),
            jax.ShapeDtypeStruct((NC, NS, RPT), jnp.float32),
        ],
        mesh=mesh,
        scratch_types=[
            pltpu.VMEM((CH,), jnp.int32),
            pltpu.VMEM((CH,), jnp.int32),
            pltpu.VMEM((CH, D_HID), jnp.float32),
            pltpu.VMEM((CH,), jnp.float32),
            pltpu.VMEM((RPT,), jnp.float32),
            pltpu.VMEM_SHARED((NPAD, D_HID), jnp.float32),
            pltpu.VMEM_SHARED((NPAD,), jnp.float32),
            pltpu.SemaphoreType.DMA,
        ],
        compiler_params=pltpu.CompilerParams(use_tc_tiling_on_sc=False),
    )(_sc_agg64_body)
    return f(ytab, srcm, dstm)


def _fuse_body(a0_ref, a1_ref, c0_ref, c1_ref, z1_ref, wl_ref, wr_ref, b2_ref,
               y2_ref, z2_ref, rc_ref):
    cnt = c0_ref[0] + c1_ref[0]
    rcnt = 1.0 / jnp.maximum(cnt, 1.0)
    h = jnp.maximum((a0_ref[0] + a1_ref[0]) * rcnt + z1_ref[...], 0.0)
    y2_ref[...] = jnp.dot(h, wl_ref[...], preferred_element_type=jnp.float32)
    z2_ref[...] = (jnp.dot(h, wr_ref[...], preferred_element_type=jnp.float32)
                   + b2_ref[...])
    rc_ref[...] = rcnt


def _phase_c(aggp, cntp3, z1, W2_l, W2_r, b2_2d):
    return pl.pallas_call(
        _fuse_body,
        grid=(NPAD // RB,),
        in_specs=[
            pl.BlockSpec((1, RB, D_HID), lambda i: (0, i, 0)),
            pl.BlockSpec((1, RB, D_HID), lambda i: (1, i, 0)),
            pl.BlockSpec((1, RB, 1), lambda i: (0, i, 0)),
            pl.BlockSpec((1, RB, 1), lambda i: (1, i, 0)),
            pl.BlockSpec((RB, D_HID), lambda i: (i, 0)),
            pl.BlockSpec((D_HID, 1), lambda i: (0, 0)),
            pl.BlockSpec((D_HID, 1), lambda i: (0, 0)),
            pl.BlockSpec((1, 1), lambda i: (0, 0)),
        ],
        out_specs=[
            pl.BlockSpec((RB, 1), lambda i: (i, 0)),
            pl.BlockSpec((RB, 1), lambda i: (i, 0)),
            pl.BlockSpec((RB, 1), lambda i: (i, 0)),
        ],
        out_shape=[
            jax.ShapeDtypeStruct((NPAD, 1), jnp.float32),
            jax.ShapeDtypeStruct((NPAD, 1), jnp.float32),
            jax.ShapeDtypeStruct((NPAD, 1), jnp.float32),
        ],
    )(aggp, aggp, cntp3, cntp3, z1, W2_l, W2_r, b2_2d)


def _sc_agg1_body(y2v, srcm, dstm, agg_out,
                  idx_s, idx_d, vals, zcnt, y2_sh, agg2_sh, sem):
    c = lax.axis_index("c")
    s = lax.axis_index("s")
    zero16 = jnp.zeros((16,), jnp.float32)

    def zcnt_body(i, carry):
        zcnt[pl.ds(i * 16, 16)] = zero16
        return carry

    lax.fori_loop(0, RPT // 16, zcnt_body, 0)

    pltpu.sync_copy(y2v.at[pl.ds(s * RPT, RPT)], y2_sh.at[pl.ds(s * RPT, RPT)])
    pltpu.sync_copy(zcnt, agg2_sh.at[pl.ds(s * RPT, RPT)])
    plsc.subcore_barrier()

    base = (c * NS + s) * CPW

    def edge_body(i, carry):
        pltpu.sync_copy(srcm.at[base + i], idx_s)
        pltpu.sync_copy(dstm.at[base + i], idx_d)
        pltpu.async_copy(y2_sh.at[idx_s], vals, sem).wait()
        pltpu.sync_copy(vals, agg2_sh.at[idx_d], add=True)
        return carry

    lax.fori_loop(0, CPW, edge_body, 0)
    plsc.subcore_barrier()

    pltpu.sync_copy(agg2_sh.at[pl.ds(s * RPT, RPT)], agg_out.at[c, s])


def _phase_d(y2v, srcm, dstm):
    mesh = plsc.VectorSubcoreMesh(core_axis_name="c", subcore_axis_name="s")
    f = functools.partial(
        pl.kernel,
        out_type=jax.ShapeDtypeStruct((NC, NS, RPT), jnp.float32),
        mesh=mesh,
        scratch_types=[
            pltpu.VMEM((CH,), jnp.int32),
            pltpu.VMEM((CH,), jnp.int32),
            pltpu.VMEM((CH,), jnp.float32),
            pltpu.VMEM((RPT,), jnp.float32),
            pltpu.VMEM_SHARED((NPAD,), jnp.float32),
            pltpu.VMEM_SHARED((NPAD,), jnp.float32),
            pltpu.SemaphoreType.DMA,
        ],
        compiler_params=pltpu.CompilerParams(use_tc_tiling_on_sc=False),
    )(_sc_agg1_body)
    return f(y2v, srcm, dstm)


def _sig_body(a0_ref, a1_ref, rc_ref, z2_ref, o_ref):
    t = (a0_ref[0] + a1_ref[0]) * rc_ref[...] + z2_ref[...]
    o_ref[...] = 1.0 / (1.0 + jnp.exp(-t))


def _phase_e(agg2p3, rcnt, z2):
    return pl.pallas_call(
        _sig_body,
        grid=(NPAD // RB,),
        in_specs=[
            pl.BlockSpec((1, RB, 1), lambda i: (0, i, 0)),
            pl.BlockSpec((1, RB, 1), lambda i: (1, i, 0)),
            pl.BlockSpec((RB, 1), lambda i: (i, 0)),
            pl.BlockSpec((RB, 1), lambda i: (i, 0)),
        ],
        out_specs=pl.BlockSpec((RB, 1), lambda i: (i, 0)),
        out_shape=jax.ShapeDtypeStruct((NPAD, 1), jnp.float32),
    )(agg2p3, agg2p3, rcnt, z2)


def kernel(x, edge_index, W1_l, b1, W1_r, W2_l, b2, W2_r):
    x = x.astype(jnp.float32)
    ei = edge_index.astype(jnp.int32)
    pad = EPAD - E
    pad_iota = jnp.arange(pad, dtype=jnp.int32)
    src_p = jnp.concatenate([ei[0], pad_iota % N])
    dst_p = jnp.concatenate([ei[1], N + pad_iota % (NPAD - N)])
    srcm = src_p.reshape(NW * CPW, CH)
    dstm = dst_p.reshape(NW * CPW, CH)
    x_pad = jnp.pad(x, ((0, NPAD - N), (0, 0)))

    ytab, z1 = _phase_a(x_pad, W1_l, W1_r, b1[None, :])
    aggp, cntp = _phase_b(ytab, srcm, dstm)
    aggp = aggp.reshape(NC, NPAD, D_HID)
    cntp3 = cntp.reshape(NC, NPAD, 1)
    y2, z2, rcnt = _phase_c(aggp, cntp3, z1, W2_l, W2_r, b2[None, :])
    agg2p = _phase_d(y2.reshape(NPAD), srcm, dstm)
    agg2p3 = agg2p.reshape(NC, NPAD, 1)
    out = _phase_e(agg2p3, rcnt, z2)
    return out[:N]


# speedup vs baseline: 17.7876x; 2.1276x over previous
import functools

import jax
import jax.numpy as jnp
from jax import lax
from jax.experimental import pallas as pl
from jax.experimental.pallas import tpu as pltpu
from jax.experimental.pallas import tpu_sc as plsc

N = 10000
E = 320000
D_IN = 128
D_HID = 64
NC = 2
NS = 16
NW = NC * NS
CH = 128
CPW = 80
NBUF = 8
NBLK = CPW // NBUF
EPAD = NW * CPW * CH
NPAD = 10240
RPT = NPAD // NS
RB = 2048


def _mm2_body(x_ref, wl_ref, wr_ref, b_ref, y_ref, z_ref):
    xb = x_ref[...]
    y_ref[...] = jnp.dot(xb, wl_ref[...], preferred_element_type=jnp.float32)
    z_ref[...] = (jnp.dot(xb, wr_ref[...], preferred_element_type=jnp.float32)
                  + b_ref[...])


def _phase_a(x_pad, W1_l, W1_r, b1_2d):
    return pl.pallas_call(
        _mm2_body,
        grid=(NPAD // RB,),
        in_specs=[
            pl.BlockSpec((RB, D_IN), lambda i: (i, 0)),
            pl.BlockSpec((D_IN, D_HID), lambda i: (0, 0)),
            pl.BlockSpec((D_IN, D_HID), lambda i: (0, 0)),
            pl.BlockSpec((1, D_HID), lambda i: (0, 0)),
        ],
        out_specs=[
            pl.BlockSpec((RB, D_HID), lambda i: (i, 0)),
            pl.BlockSpec((RB, D_HID), lambda i: (i, 0)),
        ],
        out_shape=[
            jax.ShapeDtypeStruct((NPAD, D_HID), jnp.float32),
            jax.ShapeDtypeStruct((NPAD, D_HID), jnp.float32),
        ],
    )(x_pad, W1_l, W1_r, b1_2d)


def _sc_agg64_body(ytab, srcm, dstm, agg_out, cnt_out,
                   idxs, idxd, rows, ones_v, zcnt, agg_sh, cnt_sh,
                   gsem, ssem, csem):
    c = lax.axis_index("c")
    s = lax.axis_index("s")
    zero16 = jnp.zeros((16,), jnp.float32)
    one16 = jnp.ones((16,), jnp.float32)

    def zrow_body(i, carry):
        for j in range(D_HID // 16):
            rows[0, i, pl.ds(j * 16, 16)] = zero16
        return carry

    lax.fori_loop(0, CH, zrow_body, 0)

    def fill_body(i, carry):
        ones_v[pl.ds(i * 16, 16)] = one16
        return carry

    lax.fori_loop(0, CH // 16, fill_body, 0)

    def zcnt_body(i, carry):
        zcnt[pl.ds(i * 16, 16)] = zero16
        return carry

    lax.fori_loop(0, RPT // 16, zcnt_body, 0)

    base = (c * NS + s) * CPW
    pltpu.sync_copy(srcm.at[pl.ds(base, CPW)], idxs)
    pltpu.sync_copy(dstm.at[pl.ds(base, CPW)], idxd)

    for k in range(RPT // CH):
        pltpu.sync_copy(rows.at[0], agg_sh.at[pl.ds(s * RPT + k * CH, CH)])
    pltpu.sync_copy(zcnt, cnt_sh.at[pl.ds(s * RPT, RPT)])
    plsc.subcore_barrier()

    for b in range(NBUF):
        pltpu.async_copy(ytab.at[idxs.at[b]], rows.at[b], gsem.at[b])

    def blk(g, carry):
        for b in range(NBUF):
            i = g * NBUF + b
            pltpu.make_async_copy(ytab.at[idxs.at[i]], rows.at[b],
                                  gsem.at[b]).wait()
            pltpu.async_copy(rows.at[b], agg_sh.at[idxd.at[i]], ssem.at[b],
                             add=True)
            pltpu.async_copy(ones_v, cnt_sh.at[idxd.at[i]], csem.at[b],
                             add=True)
        for b in range(NBUF):
            i = g * NBUF + b
            j = i + NBUF
            pltpu.make_async_copy(rows.at[b], agg_sh.at[idxd.at[i]],
                                  ssem.at[b]).wait()
            pltpu.make_async_copy(ones_v, cnt_sh.at[idxd.at[i]],
                                  csem.at[b]).wait()
            pltpu.async_copy(ytab.at[idxs.at[j]], rows.at[b], gsem.at[b])
        return carry

    lax.fori_loop(0, NBLK - 1, blk, 0)

    g_last = NBLK - 1
    for b in range(NBUF):
        i = g_last * NBUF + b
        pltpu.make_async_copy(ytab.at[idxs.at[i]], rows.at[b],
                              gsem.at[b]).wait()
        pltpu.async_copy(rows.at[b], agg_sh.at[idxd.at[i]], ssem.at[b],
                         add=True)
        pltpu.async_copy(ones_v, cnt_sh.at[idxd.at[i]], csem.at[b], add=True)
    for b in range(NBUF):
        i = g_last * NBUF + b
        pltpu.make_async_copy(rows.at[b], agg_sh.at[idxd.at[i]],
                              ssem.at[b]).wait()
        pltpu.make_async_copy(ones_v, cnt_sh.at[idxd.at[i]],
                              csem.at[b]).wait()
    plsc.subcore_barrier()

    pltpu.sync_copy(agg_sh.at[pl.ds(s * RPT, RPT)], agg_out.at[c, s])
    pltpu.sync_copy(cnt_sh.at[pl.ds(s * RPT, RPT)], cnt_out.at[c, s])


def _phase_b(ytab, srcm, dstm):
    mesh = plsc.VectorSubcoreMesh(core_axis_name="c", subcore_axis_name="s")
    f = functools.partial(
        pl.kernel,
        out_type=[
            jax.ShapeDtypeStruct((NC, NS, RPT, D_HID), jnp.float32),
            jax.ShapeDtypeStruct((NC, NS, RPT), jnp.float32),
        ],
        mesh=mesh,
        scratch_types=[
            pltpu.VMEM((CPW, CH), jnp.int32),
            pltpu.VMEM((CPW, CH), jnp.int32),
            pltpu.VMEM((NBUF, CH, D_HID), jnp.float32),
            pltpu.VMEM((CH,), jnp.float32),
            pltpu.VMEM((RPT,), jnp.float32),
            pltpu.VMEM_SHARED((NPAD, D_HID), jnp.float32),
            pltpu.VMEM_SHARED((NPAD,), jnp.float32),
            pltpu.SemaphoreType.DMA((NBUF,)),
            pltpu.SemaphoreType.DMA((NBUF,)),
            pltpu.SemaphoreType.DMA((NBUF,)),
        ],
        compiler_params=pltpu.CompilerParams(use_tc_tiling_on_sc=False),
    )(_sc_agg64_body)
    return f(ytab, srcm, dstm)


def _fuse_body(a0_ref, a1_ref, c0_ref, c1_ref, z1_ref, wl_ref, wr_ref, b2_ref,
               y2_ref, z2_ref, rc_ref):
    cnt = c0_ref[0] + c1_ref[0]
    rcnt = 1.0 / jnp.maximum(cnt, 1.0)
    h = jnp.maximum((a0_ref[0] + a1_ref[0]) * rcnt + z1_ref[...], 0.0)
    y2_ref[...] = jnp.dot(h, wl_ref[...], preferred_element_type=jnp.float32)
    z2_ref[...] = (jnp.dot(h, wr_ref[...], preferred_element_type=jnp.float32)
                   + b2_ref[...])
    rc_ref[...] = rcnt


def _phase_c(aggp, cntp3, z1, W2_l, W2_r, b2_2d):
    return pl.pallas_call(
        _fuse_body,
        grid=(NPAD // RB,),
        in_specs=[
            pl.BlockSpec((1, RB, D_HID), lambda i: (0, i, 0)),
            pl.BlockSpec((1, RB, D_HID), lambda i: (1, i, 0)),
            pl.BlockSpec((1, RB, 1), lambda i: (0, i, 0)),
            pl.BlockSpec((1, RB, 1), lambda i: (1, i, 0)),
            pl.BlockSpec((RB, D_HID), lambda i: (i, 0)),
            pl.BlockSpec((D_HID, 1), lambda i: (0, 0)),
            pl.BlockSpec((D_HID, 1), lambda i: (0, 0)),
            pl.BlockSpec((1, 1), lambda i: (0, 0)),
        ],
        out_specs=[
            pl.BlockSpec((RB, 1), lambda i: (i, 0)),
            pl.BlockSpec((RB, 1), lambda i: (i, 0)),
            pl.BlockSpec((RB, 1), lambda i: (i, 0)),
        ],
        out_shape=[
            jax.ShapeDtypeStruct((NPAD, 1), jnp.float32),
            jax.ShapeDtypeStruct((NPAD, 1), jnp.float32),
            jax.ShapeDtypeStruct((NPAD, 1), jnp.float32),
        ],
    )(aggp, aggp, cntp3, cntp3, z1, W2_l, W2_r, b2_2d)


def _sc_agg1_body(y2v, srcm, dstm, agg_out,
                  idxs, idxd, vals, zcnt, y2_sh, agg2_sh, gsem, ssem):
    c = lax.axis_index("c")
    s = lax.axis_index("s")
    zero16 = jnp.zeros((16,), jnp.float32)

    def zcnt_body(i, carry):
        zcnt[pl.ds(i * 16, 16)] = zero16
        return carry

    lax.fori_loop(0, RPT // 16, zcnt_body, 0)

    base = (c * NS + s) * CPW
    pltpu.sync_copy(srcm.at[pl.ds(base, CPW)], idxs)
    pltpu.sync_copy(dstm.at[pl.ds(base, CPW)], idxd)

    pltpu.sync_copy(y2v.at[pl.ds(s * RPT, RPT)], y2_sh.at[pl.ds(s * RPT, RPT)])
    pltpu.sync_copy(zcnt, agg2_sh.at[pl.ds(s * RPT, RPT)])
    plsc.subcore_barrier()

    for b in range(NBUF):
        pltpu.async_copy(y2_sh.at[idxs.at[b]], vals.at[b], gsem.at[b])

    def blk(g, carry):
        for b in range(NBUF):
            i = g * NBUF + b
            pltpu.make_async_copy(y2_sh.at[idxs.at[i]], vals.at[b],
                                  gsem.at[b]).wait()
            pltpu.async_copy(vals.at[b], agg2_sh.at[idxd.at[i]], ssem.at[b],
                             add=True)
        for b in range(NBUF):
            i = g * NBUF + b
            pltpu.make_async_copy(vals.at[b], agg2_sh.at[idxd.at[i]],
                                  ssem.at[b]).wait()
            pltpu.async_copy(y2_sh.at[idxs.at[i + NBUF]], vals.at[b],
                             gsem.at[b])
        return carry

    lax.fori_loop(0, NBLK - 1, blk, 0)

    g_last = NBLK - 1
    for b in range(NBUF):
        i = g_last * NBUF + b
        pltpu.make_async_copy(y2_sh.at[idxs.at[i]], vals.at[b],
                              gsem.at[b]).wait()
        pltpu.async_copy(vals.at[b], agg2_sh.at[idxd.at[i]], ssem.at[b],
                         add=True)
    for b in range(NBUF):
        i = g_last * NBUF + b
        pltpu.make_async_copy(vals.at[b], agg2_sh.at[idxd.at[i]],
                              ssem.at[b]).wait()
    plsc.subcore_barrier()

    pltpu.sync_copy(agg2_sh.at[pl.ds(s * RPT, RPT)], agg_out.at[c, s])


def _phase_d(y2v, srcm, dstm):
    mesh = plsc.VectorSubcoreMesh(core_axis_name="c", subcore_axis_name="s")
    f = functools.partial(
        pl.kernel,
        out_type=jax.ShapeDtypeStruct((NC, NS, RPT), jnp.float32),
        mesh=mesh,
        scratch_types=[
            pltpu.VMEM((CPW, CH), jnp.int32),
            pltpu.VMEM((CPW, CH), jnp.int32),
            pltpu.VMEM((NBUF, CH), jnp.float32),
            pltpu.VMEM((RPT,), jnp.float32),
            pltpu.VMEM_SHARED((NPAD,), jnp.float32),
            pltpu.VMEM_SHARED((NPAD,), jnp.float32),
            pltpu.SemaphoreType.DMA((NBUF,)),
            pltpu.SemaphoreType.DMA((NBUF,)),
        ],
        compiler_params=pltpu.CompilerParams(use_tc_tiling_on_sc=False),
    )(_sc_agg1_body)
    return f(y2v, srcm, dstm)


def _sig_body(a0_ref, a1_ref, rc_ref, z2_ref, o_ref):
    t = (a0_ref[0] + a1_ref[0]) * rc_ref[...] + z2_ref[...]
    o_ref[...] = 1.0 / (1.0 + jnp.exp(-t))


def _phase_e(agg2p3, rcnt, z2):
    return pl.pallas_call(
        _sig_body,
        grid=(NPAD // RB,),
        in_specs=[
            pl.BlockSpec((1, RB, 1), lambda i: (0, i, 0)),
            pl.BlockSpec((1, RB, 1), lambda i: (1, i, 0)),
            pl.BlockSpec((RB, 1), lambda i: (i, 0)),
            pl.BlockSpec((RB, 1), lambda i: (i, 0)),
        ],
        out_specs=pl.BlockSpec((RB, 1), lambda i: (i, 0)),
        out_shape=jax.ShapeDtypeStruct((NPAD, 1), jnp.float32),
    )(agg2p3, agg2p3, rcnt, z2)


def kernel(x, edge_index, W1_l, b1, W1_r, W2_l, b2, W2_r):
    x = x.astype(jnp.float32)
    ei = edge_index.astype(jnp.int32)
    pad = EPAD - E
    pad_iota = jnp.arange(pad, dtype=jnp.int32)
    src_p = jnp.concatenate([ei[0], pad_iota % N])
    dst_p = jnp.concatenate([ei[1], N + pad_iota % (NPAD - N)])
    srcm = src_p.reshape(NW * CPW, CH)
    dstm = dst_p.reshape(NW * CPW, CH)
    x_pad = jnp.pad(x, ((0, NPAD - N), (0, 0)))

    ytab, z1 = _phase_a(x_pad, W1_l, W1_r, b1[None, :])
    aggp, cntp = _phase_b(ytab, srcm, dstm)
    aggp = aggp.reshape(NC, NPAD, D_HID)
    cntp3 = cntp.reshape(NC, NPAD, 1)
    y2, z2, rcnt = _phase_c(aggp, cntp3, z1, W2_l, W2_r, b2[None, :])
    agg2p = _phase_d(y2.reshape(NPAD), srcm, dstm)
    agg2p3 = agg2p.reshape(NC, NPAD, 1)
    out = _phase_e(agg2p3, rcnt, z2)
    return out[:N]


# speedup vs baseline: 20.7746x; 1.1679x over previous
import functools

import jax
import jax.numpy as jnp
from jax import lax
from jax.experimental import pallas as pl
from jax.experimental.pallas import tpu as pltpu
from jax.experimental.pallas import tpu_sc as plsc

N = 10000
E = 320000
D_IN = 128
D_HID = 64
NC = 2
NS = 16
NW = NC * NS
CH = 128
CPW = 80
NBUF = 8
NBLK = CPW // NBUF
EPAD = NW * CPW * CH
NPAD = 10240
RPT = NPAD // NS
RB = 2048


def _mm2_body(x_ref, wl_ref, wr_ref, b_ref, y_ref, z_ref):
    xb = x_ref[...]
    y_ref[...] = jnp.dot(xb, wl_ref[...], preferred_element_type=jnp.float32)
    z_ref[...] = (jnp.dot(xb, wr_ref[...], preferred_element_type=jnp.float32)
                  + b_ref[...])


def _phase_a(x_pad, W1_l, W1_r, b1_2d):
    return pl.pallas_call(
        _mm2_body,
        grid=(NPAD // RB,),
        in_specs=[
            pl.BlockSpec((RB, D_IN), lambda i: (i, 0)),
            pl.BlockSpec((D_IN, D_HID), lambda i: (0, 0)),
            pl.BlockSpec((D_IN, D_HID), lambda i: (0, 0)),
            pl.BlockSpec((1, D_HID), lambda i: (0, 0)),
        ],
        out_specs=[
            pl.BlockSpec((RB, D_HID), lambda i: (i, 0)),
            pl.BlockSpec((RB, D_HID), lambda i: (i, 0)),
        ],
        out_shape=[
            jax.ShapeDtypeStruct((NPAD, D_HID), jnp.float32),
            jax.ShapeDtypeStruct((NPAD, D_HID), jnp.float32),
        ],
    )(x_pad, W1_l, W1_r, b1_2d)


def _sc_agg64_body(ytab, srcm, dstm, agg_out, cnt_out,
                   idxs, idxd, rows, ones_v, zcnt, agg_sh, cnt_sh,
                   gsem, ssem, csem):
    c = lax.axis_index("c")
    s = lax.axis_index("s")
    zero16 = jnp.zeros((16,), jnp.float32)
    one16 = jnp.ones((16,), jnp.float32)

    def zrow_body(i, carry):
        for j in range(D_HID // 16):
            rows[0, i, pl.ds(j * 16, 16)] = zero16
        return carry

    lax.fori_loop(0, CH, zrow_body, 0)

    def fill_body(i, carry):
        ones_v[pl.ds(i * 16, 16)] = one16
        return carry

    lax.fori_loop(0, CH // 16, fill_body, 0)

    def zcnt_body(i, carry):
        zcnt[pl.ds(i * 16, 16)] = zero16
        return carry

    lax.fori_loop(0, RPT // 16, zcnt_body, 0)

    base = (c * NS + s) * CPW
    pltpu.sync_copy(srcm.at[pl.ds(base, CPW)], idxs)
    pltpu.sync_copy(dstm.at[pl.ds(base, CPW)], idxd)

    for k in range(RPT // CH):
        pltpu.sync_copy(rows.at[0], agg_sh.at[pl.ds(s * RPT + k * CH, CH)])
    pltpu.sync_copy(zcnt, cnt_sh.at[pl.ds(s * RPT, RPT)])
    plsc.subcore_barrier()

    for b in range(NBUF):
        pltpu.async_copy(ytab.at[idxs.at[b]], rows.at[b], gsem.at[b])

    def blk(g, carry):
        for b in range(NBUF):
            i = g * NBUF + b
            pltpu.make_async_copy(ytab.at[idxs.at[i]], rows.at[b],
                                  gsem.at[b]).wait()
            pltpu.async_copy(rows.at[b], agg_sh.at[idxd.at[i]], ssem.at[b],
                             add=True)
            pltpu.async_copy(ones_v, cnt_sh.at[idxd.at[i]], csem.at[b],
                             add=True)
        for b in range(NBUF):
            i = g * NBUF + b
            j = i + NBUF
            pltpu.make_async_copy(rows.at[b], agg_sh.at[idxd.at[i]],
                                  ssem.at[b]).wait()
            pltpu.make_async_copy(ones_v, cnt_sh.at[idxd.at[i]],
                                  csem.at[b]).wait()
            pltpu.async_copy(ytab.at[idxs.at[j]], rows.at[b], gsem.at[b])
        return carry

    lax.fori_loop(0, NBLK - 1, blk, 0)

    g_last = NBLK - 1
    for b in range(NBUF):
        i = g_last * NBUF + b
        pltpu.make_async_copy(ytab.at[idxs.at[i]], rows.at[b],
                              gsem.at[b]).wait()
        pltpu.async_copy(rows.at[b], agg_sh.at[idxd.at[i]], ssem.at[b],
                         add=True)
        pltpu.async_copy(ones_v, cnt_sh.at[idxd.at[i]], csem.at[b], add=True)
    for b in range(NBUF):
        i = g_last * NBUF + b
        pltpu.make_async_copy(rows.at[b], agg_sh.at[idxd.at[i]],
                              ssem.at[b]).wait()
        pltpu.make_async_copy(ones_v, cnt_sh.at[idxd.at[i]],
                              csem.at[b]).wait()
    plsc.subcore_barrier()

    pltpu.sync_copy(agg_sh.at[pl.ds(s * RPT, RPT)],
                    agg_out.at[c, pl.ds(s * RPT, RPT)])
    pltpu.sync_copy(cnt_sh.at[pl.ds(s * RPT, RPT)],
                    cnt_out.at[c, pl.ds(s * RPT, RPT)])


def _phase_b(ytab, srcm, dstm):
    mesh = plsc.VectorSubcoreMesh(core_axis_name="c", subcore_axis_name="s")
    f = functools.partial(
        pl.kernel,
        out_type=[
            jax.ShapeDtypeStruct((NC, NPAD, D_HID), jnp.float32),
            jax.ShapeDtypeStruct((NC, NPAD), jnp.float32),
        ],
        mesh=mesh,
        scratch_types=[
            pltpu.VMEM((CPW, CH), jnp.int32),
            pltpu.VMEM((CPW, CH), jnp.int32),
            pltpu.VMEM((NBUF, CH, D_HID), jnp.float32),
            pltpu.VMEM((CH,), jnp.float32),
            pltpu.VMEM((RPT,), jnp.float32),
            pltpu.VMEM_SHARED((NPAD, D_HID), jnp.float32),
            pltpu.VMEM_SHARED((NPAD,), jnp.float32),
            pltpu.SemaphoreType.DMA((NBUF,)),
            pltpu.SemaphoreType.DMA((NBUF,)),
            pltpu.SemaphoreType.DMA((NBUF,)),
        ],
        compiler_params=pltpu.CompilerParams(use_tc_tiling_on_sc=False),
    )(_sc_agg64_body)
    return f(ytab, srcm, dstm)


def _proj_body(aggp_ref, cntp_ref, z1_ref, wl_ref, wr_ref,
               y2_ref, z2_ref, rc_ref):
    agg = aggp_ref[0] + aggp_ref[1]
    cnt = cntp_ref[0] + cntp_ref[1]
    rc = 1.0 / jnp.maximum(cnt, 1.0)
    h = jnp.maximum(agg * rc + z1_ref[...], 0.0)
    y2_ref[...] = jnp.dot(h, wl_ref[...], preferred_element_type=jnp.float32)
    z2_ref[...] = jnp.dot(h, wr_ref[...], preferred_element_type=jnp.float32)
    rc_ref[...] = rc


def _phase_c(aggp, cntp3, z1, wl, wr):
    return pl.pallas_call(
        _proj_body,
        grid=(NPAD // RB,),
        in_specs=[
            pl.BlockSpec((NC, RB, D_HID), lambda i: (0, i, 0)),
            pl.BlockSpec((NC, RB, 1), lambda i: (0, i, 0)),
            pl.BlockSpec((RB, D_HID), lambda i: (i, 0)),
            pl.BlockSpec((D_HID, 1), lambda i: (0, 0)),
            pl.BlockSpec((D_HID, 1), lambda i: (0, 0)),
        ],
        out_specs=[
            pl.BlockSpec((RB, 1), lambda i: (i, 0)),
            pl.BlockSpec((RB, 1), lambda i: (i, 0)),
            pl.BlockSpec((RB, 1), lambda i: (i, 0)),
        ],
        out_shape=[
            jax.ShapeDtypeStruct((NPAD, 1), jnp.float32),
            jax.ShapeDtypeStruct((NPAD, 1), jnp.float32),
            jax.ShapeDtypeStruct((NPAD, 1), jnp.float32),
        ],
    )(aggp, cntp3, z1, wl, wr)


def _sc_agg1_body(y2v, srcm, dstm, agg_out,
                  idxs, idxd, vals, zcnt, y2_sh, agg2_sh, gsem, ssem):
    c = lax.axis_index("c")
    s = lax.axis_index("s")
    zero16 = jnp.zeros((16,), jnp.float32)

    def zcnt_body(i, carry):
        zcnt[pl.ds(i * 16, 16)] = zero16
        return carry

    lax.fori_loop(0, RPT // 16, zcnt_body, 0)

    base = (c * NS + s) * CPW
    pltpu.sync_copy(srcm.at[pl.ds(base, CPW)], idxs)
    pltpu.sync_copy(dstm.at[pl.ds(base, CPW)], idxd)

    pltpu.sync_copy(y2v.at[pl.ds(s * RPT, RPT)], y2_sh.at[pl.ds(s * RPT, RPT)])
    pltpu.sync_copy(zcnt, agg2_sh.at[pl.ds(s * RPT, RPT)])
    plsc.subcore_barrier()

    for b in range(NBUF):
        pltpu.async_copy(y2_sh.at[idxs.at[b]], vals.at[b], gsem.at[b])

    def blk(g, carry):
        for b in range(NBUF):
            i = g * NBUF + b
            pltpu.make_async_copy(y2_sh.at[idxs.at[i]], vals.at[b],
                                  gsem.at[b]).wait()
            pltpu.async_copy(vals.at[b], agg2_sh.at[idxd.at[i]], ssem.at[b],
                             add=True)
        for b in range(NBUF):
            i = g * NBUF + b
            pltpu.make_async_copy(vals.at[b], agg2_sh.at[idxd.at[i]],
                                  ssem.at[b]).wait()
            pltpu.async_copy(y2_sh.at[idxs.at[i + NBUF]], vals.at[b],
                             gsem.at[b])
        return carry

    lax.fori_loop(0, NBLK - 1, blk, 0)

    g_last = NBLK - 1
    for b in range(NBUF):
        i = g_last * NBUF + b
        pltpu.make_async_copy(y2_sh.at[idxs.at[i]], vals.at[b],
                              gsem.at[b]).wait()
        pltpu.async_copy(vals.at[b], agg2_sh.at[idxd.at[i]], ssem.at[b],
                         add=True)
    for b in range(NBUF):
        i = g_last * NBUF + b
        pltpu.make_async_copy(vals.at[b], agg2_sh.at[idxd.at[i]],
                              ssem.at[b]).wait()
    plsc.subcore_barrier()

    pltpu.sync_copy(agg2_sh.at[pl.ds(s * RPT, RPT)],
                    agg_out.at[c, pl.ds(s * RPT, RPT)])


def _phase_d(y2v, srcm, dstm):
    mesh = plsc.VectorSubcoreMesh(core_axis_name="c", subcore_axis_name="s")
    f = functools.partial(
        pl.kernel,
        out_type=jax.ShapeDtypeStruct((NC, NPAD), jnp.float32),
        mesh=mesh,
        scratch_types=[
            pltpu.VMEM((CPW, CH), jnp.int32),
            pltpu.VMEM((CPW, CH), jnp.int32),
            pltpu.VMEM((NBUF, CH), jnp.float32),
            pltpu.VMEM((RPT,), jnp.float32),
            pltpu.VMEM_SHARED((NPAD,), jnp.float32),
            pltpu.VMEM_SHARED((NPAD,), jnp.float32),
            pltpu.SemaphoreType.DMA((NBUF,)),
            pltpu.SemaphoreType.DMA((NBUF,)),
        ],
        compiler_params=pltpu.CompilerParams(use_tc_tiling_on_sc=False),
    )(_sc_agg1_body)
    return f(y2v, srcm, dstm)


NR = NPAD // 128


def _sig_body(a_ref, rc_ref, z2_ref, b2_ref, o_ref):
    t = ((a_ref[0] + a_ref[1]) * rc_ref[...] + z2_ref[...]
         + b2_ref[0, 0])
    o_ref[...] = 1.0 / (1.0 + jnp.exp(-t))


def _phase_e(agg2p, rcnt, z2, b2_2d):
    return pl.pallas_call(
        _sig_body,
        in_specs=[
            pl.BlockSpec((NC, NR, 128), lambda: (0, 0, 0)),
            pl.BlockSpec((NR, 128), lambda: (0, 0)),
            pl.BlockSpec((NR, 128), lambda: (0, 0)),
            pl.BlockSpec((1, 1), lambda: (0, 0)),
        ],
        out_specs=pl.BlockSpec((NR, 128), lambda: (0, 0)),
        out_shape=jax.ShapeDtypeStruct((NR, 128), jnp.float32),
    )(agg2p, rcnt, z2, b2_2d)


def kernel(x, edge_index, W1_l, b1, W1_r, W2_l, b2, W2_r):
    x = x.astype(jnp.float32)
    ei = edge_index.astype(jnp.int32)
    pad = EPAD - E
    pad_iota = jnp.arange(pad, dtype=jnp.int32)
    src_p = jnp.concatenate([ei[0], pad_iota % N])
    dst_p = jnp.concatenate([ei[1], N + pad_iota % (NPAD - N)])
    srcm = src_p.reshape(NW * CPW, CH)
    dstm = dst_p.reshape(NW * CPW, CH)
    x_pad = jnp.pad(x, ((0, NPAD - N), (0, 0)))

    ytab, z1 = _phase_a(x_pad, W1_l, W1_r, b1[None, :])
    aggp, cntp = _phase_b(ytab, srcm, dstm)
    y2, z2, rcnt = _phase_c(aggp, cntp.reshape(NC, NPAD, 1), z1, W2_l, W2_r)
    y2 = y2.reshape(NPAD)
    agg2p = _phase_d(y2, srcm, dstm)
    out = _phase_e(agg2p.reshape(NC, NR, 128), rcnt.reshape(NR, 128),
                   z2.reshape(NR, 128), b2[None, :])
    return out.reshape(NPAD, 1)[:N]


# speedup vs baseline: 21.2223x; 1.0216x over previous
import functools

import jax
import jax.numpy as jnp
from jax import lax
from jax.experimental import pallas as pl
from jax.experimental.pallas import tpu as pltpu
from jax.experimental.pallas import tpu_sc as plsc

N = 10000
E = 320000
D_IN = 128
D_HID = 64
NC = 2
NS = 16
NW = NC * NS
CH = 128
CPW = 80
NBUF = 8
NBLK = CPW // NBUF
EPAD = NW * CPW * CH
NPAD = 10240
RPT = NPAD // NS
RB = 2048


def _mm2_body(x_ref, wl_ref, wr_ref, b_ref, y_ref, z_ref):
    xb = x_ref[...]
    y_ref[...] = jnp.dot(xb, wl_ref[...], preferred_element_type=jnp.float32)
    z_ref[...] = (jnp.dot(xb, wr_ref[...], preferred_element_type=jnp.float32)
                  + b_ref[...])


def _phase_a(x_pad, W1_l, W1_r, b1_2d):
    return pl.pallas_call(
        _mm2_body,
        grid=(NPAD // RB,),
        in_specs=[
            pl.BlockSpec((RB, D_IN), lambda i: (i, 0)),
            pl.BlockSpec((D_IN, D_HID), lambda i: (0, 0)),
            pl.BlockSpec((D_IN, D_HID), lambda i: (0, 0)),
            pl.BlockSpec((1, D_HID), lambda i: (0, 0)),
        ],
        out_specs=[
            pl.BlockSpec((RB, D_HID), lambda i: (i, 0)),
            pl.BlockSpec((RB, D_HID), lambda i: (i, 0)),
        ],
        out_shape=[
            jax.ShapeDtypeStruct((NPAD, D_HID), jnp.float32),
            jax.ShapeDtypeStruct((NPAD, D_HID), jnp.float32),
        ],
    )(x_pad, W1_l, W1_r, b1_2d)


def _sc_agg64_body(ytab, srcm, dstm, agg_out, cnt_out,
                   idxs, idxd, rows, ones_v, zcnt, agg_sh, cnt_sh,
                   gsem, ssem, csem):
    c = lax.axis_index("c")
    s = lax.axis_index("s")
    zero16 = jnp.zeros((16,), jnp.float32)
    one16 = jnp.ones((16,), jnp.float32)

    def zrow_body(i, carry):
        for j in range(D_HID // 16):
            rows[0, i, pl.ds(j * 16, 16)] = zero16
        return carry

    lax.fori_loop(0, CH, zrow_body, 0)

    def fill_body(i, carry):
        ones_v[pl.ds(i * 16, 16)] = one16
        return carry

    lax.fori_loop(0, CH // 16, fill_body, 0)

    def zcnt_body(i, carry):
        zcnt[pl.ds(i * 16, 16)] = zero16
        return carry

    lax.fori_loop(0, RPT // 16, zcnt_body, 0)

    base = (c * NS + s) * CPW
    pltpu.sync_copy(srcm.at[pl.ds(base, CPW)], idxs)
    pltpu.sync_copy(dstm.at[pl.ds(base, CPW)], idxd)

    for k in range(RPT // CH):
        pltpu.sync_copy(rows.at[0], agg_sh.at[pl.ds(s * RPT + k * CH, CH)])
    pltpu.sync_copy(zcnt, cnt_sh.at[pl.ds(s * RPT, RPT)])
    plsc.subcore_barrier()

    for b in range(NBUF):
        pltpu.async_copy(ytab.at[idxs.at[b]], rows.at[b], gsem.at[b])

    def blk(g, carry):
        for b in range(NBUF):
            i = g * NBUF + b
            pltpu.make_async_copy(ytab.at[idxs.at[i]], rows.at[b],
                                  gsem.at[b]).wait()
            pltpu.async_copy(rows.at[b], agg_sh.at[idxd.at[i]], ssem.at[b],
                             add=True)
            pltpu.async_copy(ones_v, cnt_sh.at[idxd.at[i]], csem.at[b],
                             add=True)
        for b in range(NBUF):
            i = g * NBUF + b
            j = i + NBUF
            pltpu.make_async_copy(rows.at[b], agg_sh.at[idxd.at[i]],
                                  ssem.at[b]).wait()
            pltpu.make_async_copy(ones_v, cnt_sh.at[idxd.at[i]],
                                  csem.at[b]).wait()
            pltpu.async_copy(ytab.at[idxs.at[j]], rows.at[b], gsem.at[b])
        return carry

    lax.fori_loop(0, NBLK - 1, blk, 0)

    g_last = NBLK - 1
    for b in range(NBUF):
        i = g_last * NBUF + b
        pltpu.make_async_copy(ytab.at[idxs.at[i]], rows.at[b],
                              gsem.at[b]).wait()
        pltpu.async_copy(rows.at[b], agg_sh.at[idxd.at[i]], ssem.at[b],
                         add=True)
        pltpu.async_copy(ones_v, cnt_sh.at[idxd.at[i]], csem.at[b], add=True)
    for b in range(NBUF):
        i = g_last * NBUF + b
        pltpu.make_async_copy(rows.at[b], agg_sh.at[idxd.at[i]],
                              ssem.at[b]).wait()
        pltpu.make_async_copy(ones_v, cnt_sh.at[idxd.at[i]],
                              csem.at[b]).wait()
    plsc.subcore_barrier()

    pltpu.sync_copy(agg_sh.at[pl.ds(s * RPT, RPT)],
                    agg_out.at[c, pl.ds(s * RPT, RPT)])
    pltpu.sync_copy(cnt_sh.at[pl.ds(s * RPT, RPT)],
                    cnt_out.at[c, pl.ds(s * RPT, RPT)])


def _phase_b(ytab, srcm, dstm):
    mesh = plsc.VectorSubcoreMesh(core_axis_name="c", subcore_axis_name="s")
    f = functools.partial(
        pl.kernel,
        out_type=[
            jax.ShapeDtypeStruct((NC, NPAD, D_HID), jnp.float32),
            jax.ShapeDtypeStruct((NC, NPAD), jnp.float32),
        ],
        mesh=mesh,
        scratch_types=[
            pltpu.VMEM((CPW, CH), jnp.int32),
            pltpu.VMEM((CPW, CH), jnp.int32),
            pltpu.VMEM((NBUF, CH, D_HID), jnp.float32),
            pltpu.VMEM((CH,), jnp.float32),
            pltpu.VMEM((RPT,), jnp.float32),
            pltpu.VMEM_SHARED((NPAD, D_HID), jnp.float32),
            pltpu.VMEM_SHARED((NPAD,), jnp.float32),
            pltpu.SemaphoreType.DMA((NBUF,)),
            pltpu.SemaphoreType.DMA((NBUF,)),
            pltpu.SemaphoreType.DMA((NBUF,)),
        ],
        compiler_params=pltpu.CompilerParams(use_tc_tiling_on_sc=False),
    )(_sc_agg64_body)
    return f(ytab, srcm, dstm)


def _proj_body(aggp_ref, cntp_ref, z1_ref, wl_ref, wr_ref,
               y2_ref, z2_ref, rc_ref):
    agg = aggp_ref[0] + aggp_ref[1]
    cnt = cntp_ref[0] + cntp_ref[1]
    rc = 1.0 / jnp.maximum(cnt, 1.0)
    h = jnp.maximum(agg * rc + z1_ref[...], 0.0)
    y2_ref[...] = jnp.dot(h, wl_ref[...], preferred_element_type=jnp.float32)
    z2_ref[...] = jnp.dot(h, wr_ref[...], preferred_element_type=jnp.float32)
    rc_ref[...] = rc


def _phase_c(aggp, cntp3, z1, wl, wr):
    return pl.pallas_call(
        _proj_body,
        grid=(NPAD // RB,),
        in_specs=[
            pl.BlockSpec((NC, RB, D_HID), lambda i: (0, i, 0)),
            pl.BlockSpec((NC, RB, 1), lambda i: (0, i, 0)),
            pl.BlockSpec((RB, D_HID), lambda i: (i, 0)),
            pl.BlockSpec((D_HID, 1), lambda i: (0, 0)),
            pl.BlockSpec((D_HID, 1), lambda i: (0, 0)),
        ],
        out_specs=[
            pl.BlockSpec((RB, 1), lambda i: (i, 0)),
            pl.BlockSpec((RB, 1), lambda i: (i, 0)),
            pl.BlockSpec((RB, 1), lambda i: (i, 0)),
        ],
        out_shape=[
            jax.ShapeDtypeStruct((NPAD, 1), jnp.float32),
            jax.ShapeDtypeStruct((NPAD, 1), jnp.float32),
            jax.ShapeDtypeStruct((NPAD, 1), jnp.float32),
        ],
    )(aggp, cntp3, z1, wl, wr)


def _sc_agg1_body(y2v, srcm, dstm, agg_out,
                  idxs, idxd, vals, zcnt, y2_sh, agg2_sh, gsem, ssem):
    c = lax.axis_index("c")
    s = lax.axis_index("s")
    zero16 = jnp.zeros((16,), jnp.float32)

    def zcnt_body(i, carry):
        zcnt[pl.ds(i * 16, 16)] = zero16
        return carry

    lax.fori_loop(0, RPT // 16, zcnt_body, 0)

    base = (c * NS + s) * CPW
    pltpu.sync_copy(srcm.at[pl.ds(base, CPW)], idxs)
    pltpu.sync_copy(dstm.at[pl.ds(base, CPW)], idxd)

    pltpu.sync_copy(y2v.at[pl.ds(s * RPT, RPT)], y2_sh.at[pl.ds(s * RPT, RPT)])
    pltpu.sync_copy(zcnt, agg2_sh.at[pl.ds(s * RPT, RPT)])
    plsc.subcore_barrier()

    for b in range(NBUF):
        pltpu.async_copy(y2_sh.at[idxs.at[b]], vals.at[b], gsem.at[b])

    def blk(g, carry):
        for b in range(NBUF):
            i = g * NBUF + b
            pltpu.make_async_copy(y2_sh.at[idxs.at[i]], vals.at[b],
                                  gsem.at[b]).wait()
            pltpu.async_copy(vals.at[b], agg2_sh.at[idxd.at[i]], ssem.at[b],
                             add=True)
        for b in range(NBUF):
            i = g * NBUF + b
            pltpu.make_async_copy(vals.at[b], agg2_sh.at[idxd.at[i]],
                                  ssem.at[b]).wait()
            pltpu.async_copy(y2_sh.at[idxs.at[i + NBUF]], vals.at[b],
                             gsem.at[b])
        return carry

    lax.fori_loop(0, NBLK - 1, blk, 0)

    g_last = NBLK - 1
    for b in range(NBUF):
        i = g_last * NBUF + b
        pltpu.make_async_copy(y2_sh.at[idxs.at[i]], vals.at[b],
                              gsem.at[b]).wait()
        pltpu.async_copy(vals.at[b], agg2_sh.at[idxd.at[i]], ssem.at[b],
                         add=True)
    for b in range(NBUF):
        i = g_last * NBUF + b
        pltpu.make_async_copy(vals.at[b], agg2_sh.at[idxd.at[i]],
                              ssem.at[b]).wait()
    plsc.subcore_barrier()

    pltpu.sync_copy(agg2_sh.at[pl.ds(s * RPT, RPT)],
                    agg_out.at[c, pl.ds(s * RPT, RPT)])


def _phase_d(y2v, srcm, dstm):
    mesh = plsc.VectorSubcoreMesh(core_axis_name="c", subcore_axis_name="s")
    f = functools.partial(
        pl.kernel,
        out_type=jax.ShapeDtypeStruct((NC, NPAD), jnp.float32),
        mesh=mesh,
        scratch_types=[
            pltpu.VMEM((CPW, CH), jnp.int32),
            pltpu.VMEM((CPW, CH), jnp.int32),
            pltpu.VMEM((NBUF, CH), jnp.float32),
            pltpu.VMEM((RPT,), jnp.float32),
            pltpu.VMEM_SHARED((NPAD,), jnp.float32),
            pltpu.VMEM_SHARED((NPAD,), jnp.float32),
            pltpu.SemaphoreType.DMA((NBUF,)),
            pltpu.SemaphoreType.DMA((NBUF,)),
        ],
        compiler_params=pltpu.CompilerParams(use_tc_tiling_on_sc=False),
    )(_sc_agg1_body)
    return f(y2v, srcm, dstm)


NR = NPAD // 128


def _sig_body(a_ref, rc_ref, z2_ref, b2_ref, o_ref):
    t = ((a_ref[0] + a_ref[1]) * rc_ref[...] + z2_ref[...]
         + b2_ref[0, 0])
    o_ref[...] = 1.0 / (1.0 + jnp.exp(-t))


def _phase_e(agg2p, rcnt, z2, b2_2d):
    return pl.pallas_call(
        _sig_body,
        in_specs=[
            pl.BlockSpec((NC, NR, 128), lambda: (0, 0, 0)),
            pl.BlockSpec((NR, 128), lambda: (0, 0)),
            pl.BlockSpec((NR, 128), lambda: (0, 0)),
            pl.BlockSpec((1, 1), lambda: (0, 0)),
        ],
        out_specs=pl.BlockSpec((NR, 128), lambda: (0, 0)),
        out_shape=jax.ShapeDtypeStruct((NR, 128), jnp.float32),
    )(agg2p, rcnt, z2, b2_2d)


def kernel(x, edge_index, W1_l, b1, W1_r, W2_l, b2, W2_r):
    x = x.astype(jnp.float32)
    ei = edge_index.astype(jnp.int32)
    flat = jnp.arange(EPAD, dtype=jnp.int32)
    inb = flat < E
    src0 = jnp.pad(ei[0], (0, EPAD - E))
    dst0 = jnp.pad(ei[1], (0, EPAD - E))
    srcm = jnp.where(inb, src0, flat % N).reshape(NW * CPW, CH)
    dstm = jnp.where(inb, dst0, N + flat % (NPAD - N)).reshape(NW * CPW, CH)

    ytab, z1 = _phase_a(x, W1_l, W1_r, b1[None, :])
    aggp, cntp = _phase_b(ytab, srcm, dstm)
    y2, z2, rcnt = _phase_c(aggp, cntp.reshape(NC, NPAD, 1), z1, W2_l, W2_r)
    y2 = y2.reshape(NPAD)
    agg2p = _phase_d(y2, srcm, dstm)
    out = _phase_e(agg2p.reshape(NC, NR, 128), rcnt.reshape(NR, 128),
                   z2.reshape(NR, 128), b2[None, :])
    return out.reshape(NPAD, 1)[:N]


# speedup vs baseline: 23.6546x; 1.1146x over previous
import functools

import jax
import jax.numpy as jnp
from jax import lax
from jax.experimental import pallas as pl
from jax.experimental.pallas import tpu as pltpu
from jax.experimental.pallas import tpu_sc as plsc

N = 10000
E = 320000
D_IN = 128
D_HID = 64
NC = 2
NS = 16
NW = NC * NS
CH = 128
CPW = 80
NBUF = 8
NBLK = CPW // NBUF
EPAD = NW * CPW * CH
NPAD = 10240
H = NPAD // 2
RPT = NPAD // NS
RB2 = 1024


def _mm2_body(xlo_ref, xhi_ref, wl_ref, wr_ref, b_ref, y_ref, z_ref):
    xlo = xlo_ref[...]
    xhi = xhi_ref[...]
    wl = wl_ref[...]
    wr = wr_ref[...]
    b = b_ref[...]
    ylo = jnp.dot(xlo, wl, preferred_element_type=jnp.float32)
    yhi = jnp.dot(xhi, wl, preferred_element_type=jnp.float32)
    zlo = jnp.dot(xlo, wr, preferred_element_type=jnp.float32) + b
    zhi = jnp.dot(xhi, wr, preferred_element_type=jnp.float32) + b
    y_ref[...] = jnp.concatenate([ylo, yhi], axis=1)
    z_ref[...] = jnp.concatenate([zlo, zhi], axis=1)


def _phase_a(x, W1_l, W1_r, b1_2d):
    return pl.pallas_call(
        _mm2_body,
        grid=(H // RB2,),
        in_specs=[
            pl.BlockSpec((RB2, D_IN), lambda i: (i, 0)),
            pl.BlockSpec((RB2, D_IN), lambda i: (i + H // RB2, 0)),
            pl.BlockSpec((D_IN, D_HID), lambda i: (0, 0)),
            pl.BlockSpec((D_IN, D_HID), lambda i: (0, 0)),
            pl.BlockSpec((1, D_HID), lambda i: (0, 0)),
        ],
        out_specs=[
            pl.BlockSpec((RB2, 2 * D_HID), lambda i: (i, 0)),
            pl.BlockSpec((RB2, 2 * D_HID), lambda i: (i, 0)),
        ],
        out_shape=[
            jax.ShapeDtypeStruct((H, 2 * D_HID), jnp.float32),
            jax.ShapeDtypeStruct((H, 2 * D_HID), jnp.float32),
        ],
    )(x, x, W1_l, W1_r, b1_2d)


def _sc_agg64_body(ytab, srcm, dstm, agg_out, cnt_out,
                   idxs, idxd, rows, ones_v, zcnt, agg_sh, cnt_sh,
                   gsem, ssem, csem):
    c = lax.axis_index("c")
    s = lax.axis_index("s")
    zero16 = jnp.zeros((16,), jnp.float32)
    one16 = jnp.ones((16,), jnp.float32)

    def zrow_body(i, carry):
        for j in range(D_HID // 16):
            rows[0, i, pl.ds(j * 16, 16)] = zero16
        return carry

    lax.fori_loop(0, CH, zrow_body, 0)

    def fill_body(i, carry):
        ones_v[pl.ds(i * 16, 16)] = one16
        return carry

    lax.fori_loop(0, CH // 16, fill_body, 0)

    def zcnt_body(i, carry):
        zcnt[pl.ds(i * 16, 16)] = zero16
        return carry

    lax.fori_loop(0, RPT // 16, zcnt_body, 0)

    base = (c * NS + s) * CPW
    pltpu.sync_copy(srcm.at[pl.ds(base, CPW)], idxs)
    pltpu.sync_copy(dstm.at[pl.ds(base, CPW)], idxd)

    for k in range(RPT // CH):
        pltpu.sync_copy(rows.at[0], agg_sh.at[pl.ds(s * RPT + k * CH, CH)])
    pltpu.sync_copy(zcnt, cnt_sh.at[pl.ds(s * RPT, RPT)])
    plsc.subcore_barrier()

    for b in range(NBUF):
        pltpu.async_copy(ytab.at[idxs.at[b]], rows.at[b], gsem.at[b])

    def blk(g, carry):
        for b in range(NBUF):
            i = g * NBUF + b
            pltpu.make_async_copy(ytab.at[idxs.at[i]], rows.at[b],
                                  gsem.at[b]).wait()
            pltpu.async_copy(rows.at[b], agg_sh.at[idxd.at[i]], ssem.at[b],
                             add=True)
            pltpu.async_copy(ones_v, cnt_sh.at[idxd.at[i]], csem.at[b],
                             add=True)
        for b in range(NBUF):
            i = g * NBUF + b
            j = i + NBUF
            pltpu.make_async_copy(rows.at[b], agg_sh.at[idxd.at[i]],
                                  ssem.at[b]).wait()
            pltpu.make_async_copy(ones_v, cnt_sh.at[idxd.at[i]],
                                  csem.at[b]).wait()
            pltpu.async_copy(ytab.at[idxs.at[j]], rows.at[b], gsem.at[b])
        return carry

    lax.fori_loop(0, NBLK - 1, blk, 0)

    g_last = NBLK - 1
    for b in range(NBUF):
        i = g_last * NBUF + b
        pltpu.make_async_copy(ytab.at[idxs.at[i]], rows.at[b],
                              gsem.at[b]).wait()
        pltpu.async_copy(rows.at[b], agg_sh.at[idxd.at[i]], ssem.at[b],
                         add=True)
        pltpu.async_copy(ones_v, cnt_sh.at[idxd.at[i]], csem.at[b], add=True)
    for b in range(NBUF):
        i = g_last * NBUF + b
        pltpu.make_async_copy(rows.at[b], agg_sh.at[idxd.at[i]],
                              ssem.at[b]).wait()
        pltpu.make_async_copy(ones_v, cnt_sh.at[idxd.at[i]],
                              csem.at[b]).wait()
    plsc.subcore_barrier()

    pltpu.sync_copy(agg_sh.at[pl.ds(s * RPT, RPT)],
                    agg_out.at[c, pl.ds(s * RPT, RPT)])
    pltpu.sync_copy(cnt_sh.at[pl.ds(s * RPT, RPT)],
                    cnt_out.at[c, pl.ds(s * RPT, RPT)])


def _phase_b(ytab, srcm, dstm):
    mesh = plsc.VectorSubcoreMesh(core_axis_name="c", subcore_axis_name="s")
    f = functools.partial(
        pl.kernel,
        out_type=[
            jax.ShapeDtypeStruct((NC, NPAD, D_HID), jnp.float32),
            jax.ShapeDtypeStruct((NC, NPAD), jnp.float32),
        ],
        mesh=mesh,
        scratch_types=[
            pltpu.VMEM((CPW, CH), jnp.int32),
            pltpu.VMEM((CPW, CH), jnp.int32),
            pltpu.VMEM((NBUF, CH, D_HID), jnp.float32),
            pltpu.VMEM((CH,), jnp.float32),
            pltpu.VMEM((RPT,), jnp.float32),
            pltpu.VMEM_SHARED((NPAD, D_HID), jnp.float32),
            pltpu.VMEM_SHARED((NPAD,), jnp.float32),
            pltpu.SemaphoreType.DMA((NBUF,)),
            pltpu.SemaphoreType.DMA((NBUF,)),
            pltpu.SemaphoreType.DMA((NBUF,)),
        ],
        compiler_params=pltpu.CompilerParams(use_tc_tiling_on_sc=False),
    )(_sc_agg64_body)
    return f(ytab, srcm, dstm)


def _proj_body(aggp_ref, cnt_ref, z1_ref, wl_ref, wr_ref,
               y2_ref, z2_ref, rc_ref):
    ag = aggp_ref[0] + aggp_ref[1]
    alo = ag[:, :D_HID]
    ahi = ag[:, D_HID:]
    zlo = z1_ref[:, :D_HID]
    zhi = z1_ref[:, D_HID:]
    csum = jnp.maximum(cnt_ref[0] + cnt_ref[1], 1.0)
    rlo = 1.0 / csum[:, 0:1]
    rhi = 1.0 / csum[:, 1:2]
    hlo = jnp.maximum(alo * rlo + zlo, 0.0)
    hhi = jnp.maximum(ahi * rhi + zhi, 0.0)
    wl = wl_ref[...]
    wr = wr_ref[...]
    y2_ref[...] = jnp.concatenate(
        [jnp.dot(hlo, wl, preferred_element_type=jnp.float32),
         jnp.dot(hhi, wl, preferred_element_type=jnp.float32)], axis=1)
    z2_ref[...] = jnp.concatenate(
        [jnp.dot(hlo, wr, preferred_element_type=jnp.float32),
         jnp.dot(hhi, wr, preferred_element_type=jnp.float32)], axis=1)
    rc_ref[...] = jnp.concatenate([rlo, rhi], axis=1)


def _phase_c(aggv, cntp3, z1p, wl, wr):
    return pl.pallas_call(
        _proj_body,
        grid=(H // RB2,),
        in_specs=[
            pl.BlockSpec((NC, RB2, 2 * D_HID), lambda i: (0, i, 0)),
            pl.BlockSpec((NC, RB2, 2), lambda i: (0, i, 0)),
            pl.BlockSpec((RB2, 2 * D_HID), lambda i: (i, 0)),
            pl.BlockSpec((D_HID, 1), lambda i: (0, 0)),
            pl.BlockSpec((D_HID, 1), lambda i: (0, 0)),
        ],
        out_specs=[
            pl.BlockSpec((RB2, 2), lambda i: (i, 0)),
            pl.BlockSpec((RB2, 2), lambda i: (i, 0)),
            pl.BlockSpec((RB2, 2), lambda i: (i, 0)),
        ],
        out_shape=[
            jax.ShapeDtypeStruct((H, 2), jnp.float32),
            jax.ShapeDtypeStruct((H, 2), jnp.float32),
            jax.ShapeDtypeStruct((H, 2), jnp.float32),
        ],
    )(aggv, cntp3, z1p, wl, wr)


def _sc_agg1_body(y2v, srcm, dstm, agg_out,
                  idxs, idxd, vals, zcnt, y2_sh, agg2_sh, gsem, ssem):
    c = lax.axis_index("c")
    s = lax.axis_index("s")
    zero16 = jnp.zeros((16,), jnp.float32)

    def zcnt_body(i, carry):
        zcnt[pl.ds(i * 16, 16)] = zero16
        return carry

    lax.fori_loop(0, RPT // 16, zcnt_body, 0)

    base = (c * NS + s) * CPW
    pltpu.sync_copy(srcm.at[pl.ds(base, CPW)], idxs)
    pltpu.sync_copy(dstm.at[pl.ds(base, CPW)], idxd)

    pltpu.sync_copy(y2v.at[pl.ds(s * RPT, RPT)], y2_sh.at[pl.ds(s * RPT, RPT)])
    pltpu.sync_copy(zcnt, agg2_sh.at[pl.ds(s * RPT, RPT)])
    plsc.subcore_barrier()

    for b in range(NBUF):
        pltpu.async_copy(y2_sh.at[idxs.at[b]], vals.at[b], gsem.at[b])

    def blk(g, carry):
        for b in range(NBUF):
            i = g * NBUF + b
            pltpu.make_async_copy(y2_sh.at[idxs.at[i]], vals.at[b],
                                  gsem.at[b]).wait()
            pltpu.async_copy(vals.at[b], agg2_sh.at[idxd.at[i]], ssem.at[b],
                             add=True)
        for b in range(NBUF):
            i = g * NBUF + b
            pltpu.make_async_copy(vals.at[b], agg2_sh.at[idxd.at[i]],
                                  ssem.at[b]).wait()
            pltpu.async_copy(y2_sh.at[idxs.at[i + NBUF]], vals.at[b],
                             gsem.at[b])
        return carry

    lax.fori_loop(0, NBLK - 1, blk, 0)

    g_last = NBLK - 1
    for b in range(NBUF):
        i = g_last * NBUF + b
        pltpu.make_async_copy(y2_sh.at[idxs.at[i]], vals.at[b],
                              gsem.at[b]).wait()
        pltpu.async_copy(vals.at[b], agg2_sh.at[idxd.at[i]], ssem.at[b],
                         add=True)
    for b in range(NBUF):
        i = g_last * NBUF + b
        pltpu.make_async_copy(vals.at[b], agg2_sh.at[idxd.at[i]],
                              ssem.at[b]).wait()
    plsc.subcore_barrier()

    pltpu.sync_copy(agg2_sh.at[pl.ds(s * RPT, RPT)],
                    agg_out.at[c, pl.ds(s * RPT, RPT)])


def _phase_d(y2v, srcm, dstm):
    mesh = plsc.VectorSubcoreMesh(core_axis_name="c", subcore_axis_name="s")
    f = functools.partial(
        pl.kernel,
        out_type=jax.ShapeDtypeStruct((NC, NPAD), jnp.float32),
        mesh=mesh,
        scratch_types=[
            pltpu.VMEM((CPW, CH), jnp.int32),
            pltpu.VMEM((CPW, CH), jnp.int32),
            pltpu.VMEM((NBUF, CH), jnp.float32),
            pltpu.VMEM((RPT,), jnp.float32),
            pltpu.VMEM_SHARED((NPAD,), jnp.float32),
            pltpu.VMEM_SHARED((NPAD,), jnp.float32),
            pltpu.SemaphoreType.DMA((NBUF,)),
            pltpu.SemaphoreType.DMA((NBUF,)),
        ],
        compiler_params=pltpu.CompilerParams(use_tc_tiling_on_sc=False),
    )(_sc_agg1_body)
    return f(y2v, srcm, dstm)


NR = NPAD // 128


def _sig_body(a_ref, rc_ref, z2_ref, b2_ref, o_ref):
    t = ((a_ref[0] + a_ref[1]) * rc_ref[...] + z2_ref[...]
         + b2_ref[0, 0])
    o_ref[...] = 1.0 / (1.0 + jnp.exp(-t))


def _phase_e(agg2p, rcnt, z2, b2_2d):
    return pl.pallas_call(
        _sig_body,
        in_specs=[
            pl.BlockSpec((NC, NR, 128), lambda: (0, 0, 0)),
            pl.BlockSpec((NR, 128), lambda: (0, 0)),
            pl.BlockSpec((NR, 128), lambda: (0, 0)),
            pl.BlockSpec((1, 1), lambda: (0, 0)),
        ],
        out_specs=pl.BlockSpec((NR, 128), lambda: (0, 0)),
        out_shape=jax.ShapeDtypeStruct((NR, 128), jnp.float32),
    )(agg2p, rcnt, z2, b2_2d)


def kernel(x, edge_index, W1_l, b1, W1_r, W2_l, b2, W2_r):
    x = x.astype(jnp.float32)
    ei = edge_index.astype(jnp.int32)
    flat = jnp.arange(EPAD, dtype=jnp.int32)
    inb = flat < E
    src_n = jnp.where(inb, jnp.pad(ei[0], (0, EPAD - E)), flat % N)
    dst_n = jnp.where(inb, jnp.pad(ei[1], (0, EPAD - E)),
                      N + flat % (NPAD - N))
    src_k = jnp.where(src_n < H, 2 * src_n, 2 * src_n - (NPAD - 1))
    dst_k = jnp.where(dst_n < H, 2 * dst_n, 2 * dst_n - (NPAD - 1))
    srcm = src_k.reshape(NW * CPW, CH)
    dstm = dst_k.reshape(NW * CPW, CH)

    ytabp, z1p = _phase_a(x, W1_l, W1_r, b1[None, :])
    aggp, cntp = _phase_b(ytabp.reshape(NPAD, D_HID), srcm, dstm)
    y2p, z2p, rcp = _phase_c(aggp.reshape(NC, H, 2 * D_HID),
                             cntp.reshape(NC, H, 2), z1p, W2_l, W2_r)
    agg2p = _phase_d(y2p.reshape(NPAD), srcm, dstm)
    out = _phase_e(agg2p.reshape(NC, NR, 128), rcp.reshape(NR, 128),
                   z2p.reshape(NR, 128), b2[None, :])
    v = out.reshape(NPAD)
    return jnp.concatenate([v[0::2], v[1::2]])[:N].reshape(N, 1)
